# Initial kernel scaffold; baseline (speedup 1.0000x reference)
import jax
import jax.numpy as jnp
from jax.experimental import pallas as pl


def kernel(x, edge_index, edge_weight, Wxz, bxz, Whz, bhz, Wxr, bxr, Whr, bhr, Wxh, bxh, Whh, bhh, Wlin, blin):
    raise NotImplementedError("write your pallas kernel here")


# speedup vs baseline: 11.0150x; 11.0150x over previous
import jax
import jax.numpy as jnp
from jax import lax
from jax.experimental import pallas as pl
from jax.experimental.pallas import tpu as pltpu
from jax.experimental.pallas import tpu_sc as plsc

N = 10000
E = 320000
F = 128
FO = 64
NC = 2
NS = 16
NW = NC * NS
EPT = E // NW
CH = 128
NCHUNK = -(-EPT // CH)
EPT_PAD = NCHUNK * CH
NPAD = 10240
DEG_SLICE = NPAD // NS
ROW_SLICE = NPAD // NS

_GDN = lax.GatherDimensionNumbers(
    offset_dims=(), collapsed_slice_dims=(0,), start_index_map=(0,))


def _bcast_lane(v, l):
    idx = jnp.full((16, 1), l, jnp.int32)
    return lax.gather(v, idx, _GDN, (1,),
                      mode=lax.GatherScatterMode.PROMISE_IN_BOUNDS)


def _deg_body(src_hbm, w_hbm, degp_hbm, srcb, wb, zb, acc):
    c = lax.axis_index("c")
    s = lax.axis_index("s")
    t = c * NS + s
    for i in range(DEG_SLICE // 16):
        zb[pl.ds(i * 16, 16)] = jnp.zeros((16,), jnp.float32)
    pltpu.sync_copy(zb, acc.at[pl.ds(s * DEG_SLICE, DEG_SLICE)])
    pltpu.sync_copy(src_hbm.at[t], srcb)
    pltpu.sync_copy(w_hbm.at[t], wb)
    plsc.subcore_barrier()

    @pl.loop(0, NCHUNK)
    def _chunk(j):
        pltpu.sync_copy(wb.at[j], acc.at[srcb.at[j]], add=True)

    plsc.subcore_barrier()
    pltpu.sync_copy(acc.at[pl.ds(s * DEG_SLICE, DEG_SLICE)],
                    degp_hbm.at[c, pl.ds(s * DEG_SLICE, DEG_SLICE)])


_deg_kernel = pl.kernel(
    _deg_body,
    out_type=jax.ShapeDtypeStruct((NC, NPAD), jnp.float32),
    mesh=plsc.VectorSubcoreMesh(core_axis_name="c", subcore_axis_name="s"),
    scratch_types=[
        pltpu.VMEM((NCHUNK, CH), jnp.int32),
        pltpu.VMEM((NCHUNK, CH), jnp.float32),
        pltpu.VMEM((DEG_SLICE,), jnp.float32),
        pltpu.VMEM_SHARED((NPAD,), jnp.float32),
    ],
)


def _scatter_body(tab_hbm, src_hbm, dst_hbm, w_hbm, pout_hbm,
                  srcj, dstj, wj, rows, zb, acc):
    c = lax.axis_index("c")
    s = lax.axis_index("s")
    t = c * NS + s
    for i in range(32):
        for k in range(F // 16):
            zb[i, pl.ds(k * 16, 16)] = jnp.zeros((16,), jnp.float32)
    for r in range(ROW_SLICE // 32):
        pltpu.sync_copy(zb, acc.at[pl.ds(s * ROW_SLICE + r * 32, 32)])
    plsc.subcore_barrier()

    @pl.loop(0, NCHUNK)
    def _chunk(j):
        pltpu.sync_copy(src_hbm.at[t, j], srcj)
        pltpu.sync_copy(dst_hbm.at[t, j], dstj)
        pltpu.sync_copy(w_hbm.at[t, j], wj)
        pltpu.sync_copy(tab_hbm.at[srcj], rows)
        for g in range(CH // 16):
            wv = wj[pl.ds(g * 16, 16)]
            for l in range(16):
                e = g * 16 + l
                coef = _bcast_lane(wv, l)
                for k in range(F // 16):
                    seg = rows[e, pl.ds(k * 16, 16)]
                    rows[e, pl.ds(k * 16, 16)] = seg * coef
        pltpu.sync_copy(rows, acc.at[dstj], add=True)

    plsc.subcore_barrier()
    pltpu.sync_copy(acc.at[pl.ds(s * ROW_SLICE, ROW_SLICE)],
                    pout_hbm.at[c, pl.ds(s * ROW_SLICE, ROW_SLICE)])


_scatter_kernel = pl.kernel(
    _scatter_body,
    out_type=jax.ShapeDtypeStruct((NC, NPAD, F), jnp.float32),
    mesh=plsc.VectorSubcoreMesh(core_axis_name="c", subcore_axis_name="s"),
    scratch_types=[
        pltpu.VMEM((CH,), jnp.int32),
        pltpu.VMEM((CH,), jnp.int32),
        pltpu.VMEM((CH,), jnp.float32),
        pltpu.VMEM((CH, F), jnp.float32),
        pltpu.VMEM((32, F), jnp.float32),
        pltpu.VMEM_SHARED((NPAD, F), jnp.float32),
    ],
)


def _dis_xs_body(degp_ref, x_ref, dis_ref, xs_ref):
    degp = degp_ref[...]
    deg = degp[0] + degp[1]
    dis = jnp.where(deg > 0, lax.rsqrt(deg), 0.0)
    dis_ref[...] = dis
    xs_ref[...] = x_ref[...] * dis[:N]


def _dis_xs(degp3, x):
    return pl.pallas_call(
        _dis_xs_body,
        out_shape=[
            jax.ShapeDtypeStruct((NPAD, 1), jnp.float32),
            jax.ShapeDtypeStruct((N, F), jnp.float32),
        ],
    )(degp3, x)


def _txys_body(p_ref, dis_ref, tx1_ref, ys_ref):
    p = p_ref[...]
    d = dis_ref[...]
    tx1 = -d * (p[0] + p[1])
    tx1_ref[...] = tx1
    ys_ref[...] = d * tx1


def _txys(p, dis10):
    blk = 1000
    return pl.pallas_call(
        _txys_body,
        grid=(N // blk,),
        in_specs=[
            pl.BlockSpec((NC, blk, F), lambda i: (0, i, 0)),
            pl.BlockSpec((blk, 1), lambda i: (i, 0)),
        ],
        out_specs=[
            pl.BlockSpec((blk, F), lambda i: (i, 0)),
            pl.BlockSpec((blk, F), lambda i: (i, 0)),
        ],
        out_shape=[
            jax.ShapeDtypeStruct((N, F), jnp.float32),
            jax.ShapeDtypeStruct((N, F), jnp.float32),
        ],
    )(p, dis10)


def _final_body(x_ref, tx1_ref, q_ref, dis_ref, Wz_ref, Wh_ref,
                bxz_ref, bhz_ref, bxh_ref, bhh_ref, Wlin_ref, blin_ref,
                o_ref):
    x = x_ref[...]
    tx1 = tx1_ref[...]
    q = q_ref[...]
    d = dis_ref[...]
    tx2 = -2.0 * d * (q[0] + q[1]) - x
    Wz = Wz_ref[...]
    Wh = Wh_ref[...]
    lz = (x @ Wz[0] + tx1 @ Wz[1] + tx2 @ Wz[2]
          + bxz_ref[...] + bhz_ref[...])
    lh = (x @ Wh[0] + tx1 @ Wh[1] + tx2 @ Wh[2]
          + bxh_ref[...] + bhh_ref[...])
    Z = jax.nn.sigmoid(lz)
    Ht = jnp.tanh(lh)
    h = jnp.maximum((1.0 - Z) * Ht, 0.0)
    o_ref[...] = h @ Wlin_ref[...] + blin_ref[...]


def _final(x, tx1, q, dis10, Wxz, Wxh, bxz, bhz, bxh, bhh, Wlin, blin):
    blk = 1000
    full2 = lambda i: (0, 0)
    full3 = lambda i: (0, 0, 0)
    return pl.pallas_call(
        _final_body,
        grid=(N // blk,),
        in_specs=[
            pl.BlockSpec((blk, F), lambda i: (i, 0)),
            pl.BlockSpec((blk, F), lambda i: (i, 0)),
            pl.BlockSpec((NC, blk, F), lambda i: (0, i, 0)),
            pl.BlockSpec((blk, 1), lambda i: (i, 0)),
            pl.BlockSpec((3, F, FO), full3),
            pl.BlockSpec((3, F, FO), full3),
            pl.BlockSpec((1, FO), full2),
            pl.BlockSpec((1, FO), full2),
            pl.BlockSpec((1, FO), full2),
            pl.BlockSpec((1, FO), full2),
            pl.BlockSpec((FO, 4), full2),
            pl.BlockSpec((1, 4), full2),
        ],
        out_specs=pl.BlockSpec((blk, 4), lambda i: (i, 0)),
        out_shape=jax.ShapeDtypeStruct((N, 4), jnp.float32),
    )(x, tx1, q, dis10, Wxz, Wxh,
      bxz.reshape(1, FO), bhz.reshape(1, FO),
      bxh.reshape(1, FO), bhh.reshape(1, FO),
      Wlin, blin.reshape(1, 4))


@jax.jit
def kernel(x, edge_index, edge_weight, Wxz, bxz, Whz, bhz, Wxr, bxr,
           Whr, bhr, Wxh, bxh, Whh, bhh, Wlin, blin):
    src = edge_index[0]
    dst = edge_index[1]

    def part(a):
        return jnp.pad(a.reshape(NW, EPT),
                       ((0, 0), (0, EPT_PAD - EPT))).reshape(NW, NCHUNK, CH)

    src3 = part(src)
    dst3 = part(dst)
    w3 = part(edge_weight)

    degp = _deg_kernel(src3, w3)
    dis, xs = _dis_xs(degp.reshape(NC, NPAD, 1), x)
    dis10 = dis[:N]
    p = _scatter_kernel(xs, src3, dst3, w3)
    tx1, ys = _txys(p, dis10)
    q = _scatter_kernel(ys, src3, dst3, w3)
    return _final(x, tx1, q, dis10, Wxz, Wxh, bxz, bhz, bxh, bhh,
                  Wlin, blin)


# speedup vs baseline: 13.9499x; 1.2664x over previous
import jax
import jax.numpy as jnp
from jax import lax
from jax.experimental import pallas as pl
from jax.experimental.pallas import tpu as pltpu
from jax.experimental.pallas import tpu_sc as plsc

N = 10000
E = 320000
F = 128
FO = 64
NC = 2
NS = 16
NW = NC * NS
EPT = E // NW
CH = 128
NCHUNK = -(-EPT // CH)
EPT_PAD = NCHUNK * CH
SCH = 64
SNCHUNK = EPT_PAD // SCH
NPAD = 10240
DEG_SLICE = NPAD // NS
ROW_SLICE = NPAD // NS

_GDN = lax.GatherDimensionNumbers(
    offset_dims=(), collapsed_slice_dims=(0,), start_index_map=(0,))


def _bcast_lane(v, l):
    idx = jnp.full((16, 1), l, jnp.int32)
    return lax.gather(v, idx, _GDN, (1,),
                      mode=lax.GatherScatterMode.PROMISE_IN_BOUNDS)


def _deg_body(src_hbm, w_hbm, degp_hbm, srcb, wb, zb, acc):
    c = lax.axis_index("c")
    s = lax.axis_index("s")
    t = c * NS + s
    for i in range(DEG_SLICE // 16):
        zb[pl.ds(i * 16, 16)] = jnp.zeros((16,), jnp.float32)
    pltpu.sync_copy(zb, acc.at[pl.ds(s * DEG_SLICE, DEG_SLICE)])
    pltpu.sync_copy(src_hbm.at[t], srcb)
    pltpu.sync_copy(w_hbm.at[t], wb)
    plsc.subcore_barrier()

    @pl.loop(0, NCHUNK)
    def _chunk(j):
        pltpu.sync_copy(wb.at[j], acc.at[srcb.at[j]], add=True)

    plsc.subcore_barrier()
    pltpu.sync_copy(acc.at[pl.ds(s * DEG_SLICE, DEG_SLICE)],
                    degp_hbm.at[c, pl.ds(s * DEG_SLICE, DEG_SLICE)])


_deg_kernel = pl.kernel(
    _deg_body,
    out_type=jax.ShapeDtypeStruct((NC, NPAD), jnp.float32),
    mesh=plsc.VectorSubcoreMesh(core_axis_name="c", subcore_axis_name="s"),
    scratch_types=[
        pltpu.VMEM((NCHUNK, CH), jnp.int32),
        pltpu.VMEM((NCHUNK, CH), jnp.float32),
        pltpu.VMEM((DEG_SLICE,), jnp.float32),
        pltpu.VMEM_SHARED((NPAD,), jnp.float32),
    ],
)


def _scatter_body(tab_hbm, pkt_hbm, pout_hbm,
                  pkt0, pkt1, rin0, rin1, rout, zb, acc,
                  sp0, sp1, sg0, sg1):
    c = lax.axis_index("c")
    s = lax.axis_index("s")
    t = c * NS + s
    for i in range(32):
        for k in range(F // 16):
            zb[i, pl.ds(k * 16, 16)] = jnp.zeros((16,), jnp.float32)
    for r in range(ROW_SLICE // 32):
        pltpu.sync_copy(zb, acc.at[pl.ds(s * ROW_SLICE + r * 32, 32)])
    plsc.subcore_barrier()

    pkts = (pkt0, pkt1)
    rins = (rin0, rin1)
    sps = (sp0, sp1)
    sgs = (sg0, sg1)

    def wait_pkt(b):
        pltpu.make_async_copy(pkt_hbm.at[t, 0], pkts[b], sps[b]).wait()

    def wait_rows(b):
        pltpu.make_async_copy(tab_hbm.at[pkts[b].at[0]], rins[b],
                              sgs[b]).wait()

    def scale_and_scatter(b):
        rin = rins[b]
        pktb = pkts[b]
        for g in range(SCH // 16):
            wv = lax.bitcast_convert_type(pktb[2, pl.ds(g * 16, 16)],
                                          jnp.float32)
            for l in range(16):
                e = g * 16 + l
                coef = _bcast_lane(wv, l)
                for k in range(F // 16):
                    rout[e, pl.ds(k * 16, 16)] = (
                        rin[e, pl.ds(k * 16, 16)] * coef)
        pltpu.sync_copy(rout, acc.at[pktb.at[1]], add=True)

    pltpu.async_copy(pkt_hbm.at[t, 0], pkt0, sp0)
    pltpu.async_copy(pkt_hbm.at[t, 1], pkt1, sp1)
    wait_pkt(0)
    pltpu.async_copy(tab_hbm.at[pkt0.at[0]], rin0, sg0)

    @pl.loop(0, SNCHUNK - 2, step=2)
    def _chunk(j0):
        for u in range(2):
            j = j0 + u
            b = u
            nb = 1 - u
            wait_pkt(nb)
            pltpu.async_copy(tab_hbm.at[pkts[nb].at[0]], rins[nb], sgs[nb])
            wait_rows(b)
            scale_and_scatter(b)
            pltpu.async_copy(pkt_hbm.at[t, j + 2], pkts[b], sps[b])

    wait_pkt(1)
    pltpu.async_copy(tab_hbm.at[pkt1.at[0]], rin1, sg1)
    wait_rows(0)
    scale_and_scatter(0)
    wait_rows(1)
    scale_and_scatter(1)

    plsc.subcore_barrier()
    pltpu.sync_copy(acc.at[pl.ds(s * ROW_SLICE, ROW_SLICE)],
                    pout_hbm.at[c, pl.ds(s * ROW_SLICE, ROW_SLICE)])


_scatter_kernel = pl.kernel(
    _scatter_body,
    out_type=jax.ShapeDtypeStruct((NC, NPAD, F), jnp.float32),
    mesh=plsc.VectorSubcoreMesh(core_axis_name="c", subcore_axis_name="s"),
    scratch_types=[
        pltpu.VMEM((3, SCH), jnp.int32),
        pltpu.VMEM((3, SCH), jnp.int32),
        pltpu.VMEM((SCH, F), jnp.float32),
        pltpu.VMEM((SCH, F), jnp.float32),
        pltpu.VMEM((SCH, F), jnp.float32),
        pltpu.VMEM((32, F), jnp.float32),
        pltpu.VMEM_SHARED((NPAD, F), jnp.float32),
        pltpu.SemaphoreType.DMA,
        pltpu.SemaphoreType.DMA,
        pltpu.SemaphoreType.DMA,
        pltpu.SemaphoreType.DMA,
    ],
)


def _dis_xs_body(degp_ref, x_ref, dis_ref, xs_ref):
    degp = degp_ref[...]
    deg = degp[0] + degp[1]
    dis = jnp.where(deg > 0, lax.rsqrt(deg), 0.0)
    dis_ref[...] = dis
    xs_ref[...] = x_ref[...] * dis[:N]


def _dis_xs(degp3, x):
    return pl.pallas_call(
        _dis_xs_body,
        out_shape=[
            jax.ShapeDtypeStruct((NPAD, 1), jnp.float32),
            jax.ShapeDtypeStruct((N, F), jnp.float32),
        ],
    )(degp3, x)


def _txys_body(p_ref, dis_ref, tx1_ref, ys_ref):
    p = p_ref[...]
    d = dis_ref[...]
    tx1 = -d * (p[0] + p[1])
    tx1_ref[...] = tx1
    ys_ref[...] = d * tx1


def _txys(p, dis10):
    blk = 1000
    return pl.pallas_call(
        _txys_body,
        grid=(N // blk,),
        in_specs=[
            pl.BlockSpec((NC, blk, F), lambda i: (0, i, 0)),
            pl.BlockSpec((blk, 1), lambda i: (i, 0)),
        ],
        out_specs=[
            pl.BlockSpec((blk, F), lambda i: (i, 0)),
            pl.BlockSpec((blk, F), lambda i: (i, 0)),
        ],
        out_shape=[
            jax.ShapeDtypeStruct((N, F), jnp.float32),
            jax.ShapeDtypeStruct((N, F), jnp.float32),
        ],
    )(p, dis10)


def _final_body(x_ref, tx1_ref, q_ref, dis_ref, Wz_ref, Wh_ref,
                bxz_ref, bhz_ref, bxh_ref, bhh_ref, Wlin_ref, blin_ref,
                o_ref):
    x = x_ref[...]
    tx1 = tx1_ref[...]
    q = q_ref[...]
    d = dis_ref[...]
    tx2 = -2.0 * d * (q[0] + q[1]) - x
    Wz = Wz_ref[...]
    Wh = Wh_ref[...]
    lz = (x @ Wz[0] + tx1 @ Wz[1] + tx2 @ Wz[2]
          + bxz_ref[...] + bhz_ref[...])
    lh = (x @ Wh[0] + tx1 @ Wh[1] + tx2 @ Wh[2]
          + bxh_ref[...] + bhh_ref[...])
    Z = jax.nn.sigmoid(lz)
    Ht = jnp.tanh(lh)
    h = jnp.maximum((1.0 - Z) * Ht, 0.0)
    o_ref[...] = h @ Wlin_ref[...] + blin_ref[...]


def _final(x, tx1, q, dis10, Wxz, Wxh, bxz, bhz, bxh, bhh, Wlin, blin):
    blk = 1000
    full2 = lambda i: (0, 0)
    full3 = lambda i: (0, 0, 0)
    return pl.pallas_call(
        _final_body,
        grid=(N // blk,),
        in_specs=[
            pl.BlockSpec((blk, F), lambda i: (i, 0)),
            pl.BlockSpec((blk, F), lambda i: (i, 0)),
            pl.BlockSpec((NC, blk, F), lambda i: (0, i, 0)),
            pl.BlockSpec((blk, 1), lambda i: (i, 0)),
            pl.BlockSpec((3, F, FO), full3),
            pl.BlockSpec((3, F, FO), full3),
            pl.BlockSpec((1, FO), full2),
            pl.BlockSpec((1, FO), full2),
            pl.BlockSpec((1, FO), full2),
            pl.BlockSpec((1, FO), full2),
            pl.BlockSpec((FO, 4), full2),
            pl.BlockSpec((1, 4), full2),
        ],
        out_specs=pl.BlockSpec((blk, 4), lambda i: (i, 0)),
        out_shape=jax.ShapeDtypeStruct((N, 4), jnp.float32),
    )(x, tx1, q, dis10, Wxz, Wxh,
      bxz.reshape(1, FO), bhz.reshape(1, FO),
      bxh.reshape(1, FO), bhh.reshape(1, FO),
      Wlin, blin.reshape(1, 4))


@jax.jit
def kernel(x, edge_index, edge_weight, Wxz, bxz, Whz, bhz, Wxr, bxr,
           Whr, bhr, Wxh, bxh, Whh, bhh, Wlin, blin):
    src = edge_index[0]
    dst = edge_index[1]

    def pad2(a):
        return jnp.pad(a.reshape(NW, EPT), ((0, 0), (0, EPT_PAD - EPT)))

    src2 = pad2(src)
    dst2 = pad2(dst)
    w2 = pad2(edge_weight)
    src3 = src2.reshape(NW, NCHUNK, CH)
    w3 = w2.reshape(NW, NCHUNK, CH)
    pkt = jnp.stack(
        [src2.reshape(NW, SNCHUNK, SCH),
         dst2.reshape(NW, SNCHUNK, SCH),
         lax.bitcast_convert_type(w2, jnp.int32).reshape(NW, SNCHUNK, SCH)],
        axis=2)

    degp = _deg_kernel(src3, w3)
    dis, xs = _dis_xs(degp.reshape(NC, NPAD, 1), x)
    dis10 = dis[:N]
    p = _scatter_kernel(xs, pkt)
    tx1, ys = _txys(p, dis10)
    q = _scatter_kernel(ys, pkt)
    return _final(x, tx1, q, dis10, Wxz, Wxh, bxz, bhz, bxh, bhh,
                  Wlin, blin)


# speedup vs baseline: 15.3843x; 1.1028x over previous
import jax
import jax.numpy as jnp
from jax import lax
from jax.experimental import pallas as pl
from jax.experimental.pallas import tpu as pltpu
from jax.experimental.pallas import tpu_sc as plsc

N = 10000
E = 320000
F = 128
FO = 64
NC = 2
NS = 16
NW = NC * NS
EPT = E // NW
CH = 128
NCHUNK = -(-EPT // CH)
EPT_PAD = NCHUNK * CH
SCH = 64
SNCHUNK = EPT_PAD // SCH
NPAD = 10240
DEG_SLICE = NPAD // NS
ROW_SLICE = NPAD // NS

_GDN = lax.GatherDimensionNumbers(
    offset_dims=(), collapsed_slice_dims=(0,), start_index_map=(0,))


def _bcast_lane(v, l):
    idx = jnp.full((16, 1), l, jnp.int32)
    return lax.gather(v, idx, _GDN, (1,),
                      mode=lax.GatherScatterMode.PROMISE_IN_BOUNDS)


def _deg_body(src_hbm, w_hbm, degp_hbm, srcb, wb, zb, acc):
    c = lax.axis_index("c")
    s = lax.axis_index("s")
    t = c * NS + s
    for i in range(DEG_SLICE // 16):
        zb[pl.ds(i * 16, 16)] = jnp.zeros((16,), jnp.float32)
    pltpu.sync_copy(zb, acc.at[pl.ds(s * DEG_SLICE, DEG_SLICE)])
    pltpu.sync_copy(src_hbm.at[t], srcb)
    pltpu.sync_copy(w_hbm.at[t], wb)
    plsc.subcore_barrier()

    @pl.loop(0, NCHUNK)
    def _chunk(j):
        pltpu.sync_copy(wb.at[j], acc.at[srcb.at[j]], add=True)

    plsc.subcore_barrier()
    pltpu.sync_copy(acc.at[pl.ds(s * DEG_SLICE, DEG_SLICE)],
                    degp_hbm.at[c, pl.ds(s * DEG_SLICE, DEG_SLICE)])


_deg_kernel = pl.kernel(
    _deg_body,
    out_type=jax.ShapeDtypeStruct((NC, NPAD), jnp.float32),
    mesh=plsc.VectorSubcoreMesh(core_axis_name="c", subcore_axis_name="s"),
    scratch_types=[
        pltpu.VMEM((NCHUNK, CH), jnp.int32),
        pltpu.VMEM((NCHUNK, CH), jnp.float32),
        pltpu.VMEM((DEG_SLICE,), jnp.float32),
        pltpu.VMEM_SHARED((NPAD,), jnp.float32),
    ],
)


def _scatter_body(tab_hbm, pkt_hbm, pout_hbm,
                  pkt0, pkt1, rin0, rin1, rout0, rout1, sidx0, sidx1, zb,
                  acc, sp0, sp1, sg0, sg1, ss0, ss1):
    c = lax.axis_index("c")
    s = lax.axis_index("s")
    t = c * NS + s
    for i in range(32):
        for k in range(F // 16):
            zb[i, pl.ds(k * 16, 16)] = jnp.zeros((16,), jnp.float32)
    for r in range(ROW_SLICE // 32):
        pltpu.sync_copy(zb, acc.at[pl.ds(s * ROW_SLICE + r * 32, 32)])
    plsc.subcore_barrier()

    pkts = (pkt0, pkt1)
    rins = (rin0, rin1)
    routs = (rout0, rout1)
    sidxs = (sidx0, sidx1)
    sps = (sp0, sp1)
    sgs = (sg0, sg1)
    sss = (ss0, ss1)

    def wait_pkt(b):
        pltpu.make_async_copy(pkt_hbm.at[t, 0], pkts[b], sps[b]).wait()

    def wait_rows(b):
        pltpu.make_async_copy(tab_hbm.at[pkts[b].at[0]], rins[b],
                              sgs[b]).wait()

    def wait_scat(b):
        pltpu.make_async_copy(routs[b], acc.at[sidxs[b]], sss[b]).wait()

    def scale(b):
        rin = rins[b]
        rout = routs[b]
        pktb = pkts[b]
        sidx = sidxs[b]
        for k in range(SCH // 16):
            sidx[pl.ds(k * 16, 16)] = pktb[1, pl.ds(k * 16, 16)]
        for g in range(SCH // 16):
            wv = lax.bitcast_convert_type(pktb[2, pl.ds(g * 16, 16)],
                                          jnp.float32)
            for l in range(16):
                e = g * 16 + l
                coef = _bcast_lane(wv, l)
                for k in range(F // 16):
                    rout[e, pl.ds(k * 16, 16)] = (
                        rin[e, pl.ds(k * 16, 16)] * coef)

    def start_scatter(b):
        pltpu.async_copy(routs[b], acc.at[sidxs[b]], sss[b], add=True)

    pltpu.async_copy(pkt_hbm.at[t, 0], pkt0, sp0)
    pltpu.async_copy(pkt_hbm.at[t, 1], pkt1, sp1)
    wait_pkt(0)
    pltpu.async_copy(tab_hbm.at[pkt0.at[0]], rin0, sg0)
    wait_pkt(1)
    pltpu.async_copy(tab_hbm.at[pkt1.at[0]], rin1, sg1)
    wait_rows(0)
    scale(0)
    start_scatter(0)
    pltpu.async_copy(pkt_hbm.at[t, 2], pkt0, sp0)
    wait_pkt(0)
    pltpu.async_copy(tab_hbm.at[pkt0.at[0]], rin0, sg0)
    wait_rows(1)
    scale(1)
    start_scatter(1)
    pltpu.async_copy(pkt_hbm.at[t, 3], pkt1, sp1)

    @pl.loop(2, SNCHUNK - 2, step=2)
    def _chunk(j0):
        for u in range(2):
            j = j0 + u
            b = u
            nb = 1 - u
            wait_pkt(nb)
            pltpu.async_copy(tab_hbm.at[pkts[nb].at[0]], rins[nb], sgs[nb])
            wait_rows(b)
            wait_scat(b)
            scale(b)
            start_scatter(b)
            pltpu.async_copy(pkt_hbm.at[t, j + 2], pkts[b], sps[b])

    wait_pkt(1)
    pltpu.async_copy(tab_hbm.at[pkt1.at[0]], rin1, sg1)
    wait_rows(0)
    wait_scat(0)
    scale(0)
    start_scatter(0)
    wait_rows(1)
    wait_scat(1)
    scale(1)
    start_scatter(1)
    wait_scat(0)
    wait_scat(1)

    plsc.subcore_barrier()
    pltpu.sync_copy(acc.at[pl.ds(s * ROW_SLICE, ROW_SLICE)],
                    pout_hbm.at[c, pl.ds(s * ROW_SLICE, ROW_SLICE)])


_scatter_kernel = pl.kernel(
    _scatter_body,
    out_type=jax.ShapeDtypeStruct((NC, NPAD, F), jnp.float32),
    mesh=plsc.VectorSubcoreMesh(core_axis_name="c", subcore_axis_name="s"),
    scratch_types=[
        pltpu.VMEM((3, SCH), jnp.int32),
        pltpu.VMEM((3, SCH), jnp.int32),
        pltpu.VMEM((SCH, F), jnp.float32),
        pltpu.VMEM((SCH, F), jnp.float32),
        pltpu.VMEM((SCH, F), jnp.float32),
        pltpu.VMEM((SCH, F), jnp.float32),
        pltpu.VMEM((SCH,), jnp.int32),
        pltpu.VMEM((SCH,), jnp.int32),
        pltpu.VMEM((32, F), jnp.float32),
        pltpu.VMEM_SHARED((NPAD, F), jnp.float32),
        pltpu.SemaphoreType.DMA,
        pltpu.SemaphoreType.DMA,
        pltpu.SemaphoreType.DMA,
        pltpu.SemaphoreType.DMA,
        pltpu.SemaphoreType.DMA,
        pltpu.SemaphoreType.DMA,
    ],
)


def _dis_xs_body(degp_ref, x_ref, dis_ref, xs_ref):
    degp = degp_ref[...]
    deg = degp[0] + degp[1]
    dis = jnp.where(deg > 0, lax.rsqrt(deg), 0.0)
    dis_ref[...] = dis
    xs_ref[...] = x_ref[...] * dis[:N]


def _dis_xs(degp3, x):
    return pl.pallas_call(
        _dis_xs_body,
        out_shape=[
            jax.ShapeDtypeStruct((NPAD, 1), jnp.float32),
            jax.ShapeDtypeStruct((N, F), jnp.float32),
        ],
    )(degp3, x)


def _txys_body(p_ref, dis_ref, tx1_ref, ys_ref):
    p = p_ref[...]
    d = dis_ref[...]
    tx1 = -d * (p[0] + p[1])
    tx1_ref[...] = tx1
    ys_ref[...] = d * tx1


def _txys(p, dis10):
    blk = 1000
    return pl.pallas_call(
        _txys_body,
        grid=(N // blk,),
        in_specs=[
            pl.BlockSpec((NC, blk, F), lambda i: (0, i, 0)),
            pl.BlockSpec((blk, 1), lambda i: (i, 0)),
        ],
        out_specs=[
            pl.BlockSpec((blk, F), lambda i: (i, 0)),
            pl.BlockSpec((blk, F), lambda i: (i, 0)),
        ],
        out_shape=[
            jax.ShapeDtypeStruct((N, F), jnp.float32),
            jax.ShapeDtypeStruct((N, F), jnp.float32),
        ],
    )(p, dis10)


def _final_body(x_ref, tx1_ref, q_ref, dis_ref, Wz_ref, Wh_ref,
                bxz_ref, bhz_ref, bxh_ref, bhh_ref, Wlin_ref, blin_ref,
                o_ref):
    x = x_ref[...]
    tx1 = tx1_ref[...]
    q = q_ref[...]
    d = dis_ref[...]
    tx2 = -2.0 * d * (q[0] + q[1]) - x
    Wz = Wz_ref[...]
    Wh = Wh_ref[...]
    lz = (x @ Wz[0] + tx1 @ Wz[1] + tx2 @ Wz[2]
          + bxz_ref[...] + bhz_ref[...])
    lh = (x @ Wh[0] + tx1 @ Wh[1] + tx2 @ Wh[2]
          + bxh_ref[...] + bhh_ref[...])
    Z = jax.nn.sigmoid(lz)
    Ht = jnp.tanh(lh)
    h = jnp.maximum((1.0 - Z) * Ht, 0.0)
    o_ref[...] = h @ Wlin_ref[...] + blin_ref[...]


def _final(x, tx1, q, dis10, Wxz, Wxh, bxz, bhz, bxh, bhh, Wlin, blin):
    blk = 1000
    full2 = lambda i: (0, 0)
    full3 = lambda i: (0, 0, 0)
    return pl.pallas_call(
        _final_body,
        grid=(N // blk,),
        in_specs=[
            pl.BlockSpec((blk, F), lambda i: (i, 0)),
            pl.BlockSpec((blk, F), lambda i: (i, 0)),
            pl.BlockSpec((NC, blk, F), lambda i: (0, i, 0)),
            pl.BlockSpec((blk, 1), lambda i: (i, 0)),
            pl.BlockSpec((3, F, FO), full3),
            pl.BlockSpec((3, F, FO), full3),
            pl.BlockSpec((1, FO), full2),
            pl.BlockSpec((1, FO), full2),
            pl.BlockSpec((1, FO), full2),
            pl.BlockSpec((1, FO), full2),
            pl.BlockSpec((FO, 4), full2),
            pl.BlockSpec((1, 4), full2),
        ],
        out_specs=pl.BlockSpec((blk, 4), lambda i: (i, 0)),
        out_shape=jax.ShapeDtypeStruct((N, 4), jnp.float32),
    )(x, tx1, q, dis10, Wxz, Wxh,
      bxz.reshape(1, FO), bhz.reshape(1, FO),
      bxh.reshape(1, FO), bhh.reshape(1, FO),
      Wlin, blin.reshape(1, 4))


@jax.jit
def kernel(x, edge_index, edge_weight, Wxz, bxz, Whz, bhz, Wxr, bxr,
           Whr, bhr, Wxh, bxh, Whh, bhh, Wlin, blin):
    src = edge_index[0]
    dst = edge_index[1]

    def pad2(a):
        return jnp.pad(a.reshape(NW, EPT), ((0, 0), (0, EPT_PAD - EPT)))

    src2 = pad2(src)
    dst2 = pad2(dst)
    w2 = pad2(edge_weight)
    src3 = src2.reshape(NW, NCHUNK, CH)
    w3 = w2.reshape(NW, NCHUNK, CH)
    pkt = jnp.stack(
        [src2.reshape(NW, SNCHUNK, SCH),
         dst2.reshape(NW, SNCHUNK, SCH),
         lax.bitcast_convert_type(w2, jnp.int32).reshape(NW, SNCHUNK, SCH)],
        axis=2)

    degp = _deg_kernel(src3, w3)
    dis, xs = _dis_xs(degp.reshape(NC, NPAD, 1), x)
    dis10 = dis[:N]
    p = _scatter_kernel(xs, pkt)
    tx1, ys = _txys(p, dis10)
    q = _scatter_kernel(ys, pkt)
    return _final(x, tx1, q, dis10, Wxz, Wxh, bxz, bhz, bxh, bhh,
                  Wlin, blin)
